# baseline (speedup 1.0000x reference)
import jax
import jax.numpy as jnp
from jax.experimental import pallas as pl
from jax.experimental.pallas import tpu as pltpu
from jax.experimental.pallas import tpu_sc as plsc


_M_BLK = 512
_GATHER_W = 128


_GROUP_B1 = 2736
_GROUP_B2 = 5472


def _argmin_body(flat_ref, wtT_ref, bt_ref, cbT_ref, idx_ref, c2_ref):
    @pl.when(pl.program_id(0) == 0)
    def _():
        cbT = cbT_ref[...]
        c2_ref[...] = jnp.sum(cbT * cbT, axis=0, keepdims=True)

    fc = jax.lax.dot_general(
        flat_ref[...], wtT_ref[...], (((1,), (0,)), ((), ())),
        preferred_element_type=jnp.float32) + bt_ref[...]
    a = jnp.sum(fc * fc, axis=1, keepdims=True)

    k_total = cbT_ref.shape[1]
    sentinel = jnp.int32(k_total)
    m = jax.lax.dot_general(
        fc, cbT_ref[...], (((1,), (0,)), ((), ())),
        preferred_element_type=jnp.float32)
    dist_full = a - 2.0 * m + c2_ref[...]
    inf = jnp.float32(jnp.inf)

    def champ_slab(lo, hi):
        dist = dist_full[:, lo:hi]
        gv = jnp.min(dist, axis=1, keepdims=True)
        ii = jax.lax.broadcasted_iota(jnp.int32, dist.shape, 1) + lo
        gidx = jnp.min(jnp.where(dist == gv, ii, sentinel), axis=1,
                       keepdims=True)
        return gv, gidx

    def champ_masked(alo, ahi, lo, hi):
        dist = dist_full[:, alo:ahi]
        ii = jax.lax.broadcasted_iota(jnp.int32, dist.shape, 1) + alo
        dist = jnp.where((ii >= lo) & (ii < hi), dist, inf)
        gv = jnp.min(dist, axis=1, keepdims=True)
        gidx = jnp.min(jnp.where(dist == gv, ii, sentinel), axis=1,
                       keepdims=True)
        return gv, gidx

    def lexmin(p, q):
        pv, pi = p
        qv, qi = q
        take_q = (qv < pv) | ((qv == pv) & (qi < pi))
        return jnp.where(take_q, qv, pv), jnp.where(take_q, qi, pi)

    b1a = _GROUP_B1 // 128 * 128
    b1b = b1a + 128
    b2a = _GROUP_B2 // 128 * 128
    b2b = b2a + 128
    v1, i1 = lexmin(champ_slab(0, b1a),
                    champ_masked(b1a, b1b, b1a, _GROUP_B1))
    v2, i2 = lexmin(lexmin(champ_masked(b1a, b1b, _GROUP_B1, b1b),
                           champ_slab(b1b, b2a)),
                    champ_masked(b2a, b2b, b2a, _GROUP_B2))
    v3, i3 = lexmin(champ_masked(b2a, b2b, _GROUP_B2, b2b),
                    champ_slab(b2b, k_total))

    t1 = v1.astype(jnp.bfloat16).astype(jnp.float32)
    take2 = (v2 < t1) | ((v2 == t1) & (i2 < i1))
    acc_v = jnp.where(take2, v2, t1)
    acc_i = jnp.where(take2, i2, i1)
    t2 = acc_v.astype(jnp.bfloat16).astype(jnp.float32)
    take3 = (v3 < t2) | ((v3 == t2) & (i3 < acc_i))
    idx_ref[...] = jnp.where(take3, i3, acc_i)


def _out_body(flat_ref, q_ref, wfT_ref, bf_ref, o_ref):
    q16 = q_ref[...].astype(jnp.bfloat16)
    qo = jax.lax.dot_general(
        q16, wfT_ref[...], (((1,), (0,)), ((), ())),
        preferred_element_type=jnp.float32) + bf_ref[...]
    fl = flat_ref[...]
    o_ref[...] = fl + (qo - fl)


def _sc_gather(table, idx2d, n, d_code):
    mesh = plsc.VectorSubcoreMesh(core_axis_name="core", subcore_axis_name="subcore")

    @pl.kernel(out_type=jax.ShapeDtypeStruct((n, d_code), table.dtype), mesh=mesh)
    def gather_kernel(x_hbm, i_hbm, o_hbm):
        def body(i_vmem, o_vmem):
            pltpu.sync_copy(x_hbm.at[i_vmem.at[0]], o_vmem)

        pltpu.emit_pipeline(
            body,
            grid=(n // _GATHER_W,),
            in_specs=[pl.BlockSpec((1, _GATHER_W), index_map=lambda i: (0, i))],
            out_specs=[pl.BlockSpec((_GATHER_W, d_code), index_map=lambda i: (i, 0))],
            core_axis_name=("core", "subcore"),
            dimension_semantics=(pltpu.PARALLEL,),
        )(i_hbm, o_hbm)

    return gather_kernel(table, idx2d)


def kernel(features, codebook, Wt, bt, Wf, bf):
    orig_shape = features.shape
    d_in = orig_shape[-1]
    flat = features.reshape(-1, d_in)
    n = flat.shape[0]
    k, d_code = codebook.shape

    wtT = Wt.T
    cbT = codebook.T
    wfT = Wf.T
    bt2 = bt.reshape(1, d_code)
    bf2 = bf.reshape(1, d_in)

    nsteps = n // _M_BLK
    idx2d = pl.pallas_call(
        _argmin_body,
        grid=(nsteps,),
        in_specs=[
            pl.BlockSpec((_M_BLK, d_in), lambda i: (i, 0)),
            pl.BlockSpec((d_in, d_code), lambda i: (0, 0)),
            pl.BlockSpec((1, d_code), lambda i: (0, 0)),
            pl.BlockSpec((d_code, k), lambda i: (0, 0)),
        ],
        out_specs=pl.BlockSpec((_M_BLK, 1), lambda i: (i, 0)),
        out_shape=jax.ShapeDtypeStruct((n, 1), jnp.int32),
        scratch_shapes=[pltpu.VMEM((1, k), jnp.float32)],
        compiler_params=pltpu.CompilerParams(
            dimension_semantics=("arbitrary",)),
    )(flat, wtT, bt2, cbT)

    indices = idx2d[:, 0]
    quantized = _sc_gather(codebook, indices.reshape(1, n), n, d_code)

    out = pl.pallas_call(
        _out_body,
        grid=(n // 1024,),
        in_specs=[
            pl.BlockSpec((1024, d_in), lambda i: (i, 0)),
            pl.BlockSpec((1024, d_code), lambda i: (i, 0)),
            pl.BlockSpec((d_code, d_in), lambda i: (0, 0)),
            pl.BlockSpec((1, d_in), lambda i: (0, 0)),
        ],
        out_specs=pl.BlockSpec((1024, d_in), lambda i: (i, 0)),
        out_shape=jax.ShapeDtypeStruct((n, d_in), jnp.float32),
        compiler_params=pltpu.CompilerParams(
            dimension_semantics=("arbitrary",)),
    )(flat, quantized, wfT, bf2)

    return out.reshape(orig_shape), indices.reshape(orig_shape[:-1])


# speedup vs baseline: 1.0183x; 1.0183x over previous
import jax
import jax.numpy as jnp
from jax.experimental import pallas as pl
from jax.experimental.pallas import tpu as pltpu
from jax.experimental.pallas import tpu_sc as plsc


_M_BLK = 512
_GATHER_W = 128


_GROUP_B1 = 2736
_GROUP_B2 = 5472


def _argmin_body(flat_ref, wtT_ref, bt_ref, cbT_ref, idx_ref, c2_ref):
    @pl.when(pl.program_id(0) == 0)
    def _():
        cbT = cbT_ref[...]
        c2_ref[...] = jnp.sum(cbT * cbT, axis=0, keepdims=True)

    fc = jax.lax.dot_general(
        flat_ref[...], wtT_ref[...], (((1,), (0,)), ((), ())),
        preferred_element_type=jnp.float32) + bt_ref[...]
    a = jnp.sum(fc * fc, axis=1, keepdims=True)

    k_total = cbT_ref.shape[1]
    sentinel = jnp.int32(k_total)
    m2 = jax.lax.dot_general(
        -2.0 * fc, cbT_ref[...], (((1,), (0,)), ((), ())),
        preferred_element_type=jnp.float32)
    dist_full = a + m2 + c2_ref[...]
    inf = jnp.float32(jnp.inf)

    def champ_slab(lo, hi):
        dist = dist_full[:, lo:hi]
        gv = jnp.min(dist, axis=1, keepdims=True)
        ii = jax.lax.broadcasted_iota(jnp.int32, dist.shape, 1) + lo
        gidx = jnp.min(jnp.where(dist == gv, ii, sentinel), axis=1,
                       keepdims=True)
        return gv, gidx

    def champ_masked(alo, ahi, lo, hi):
        dist = dist_full[:, alo:ahi]
        ii = jax.lax.broadcasted_iota(jnp.int32, dist.shape, 1) + alo
        dist = jnp.where((ii >= lo) & (ii < hi), dist, inf)
        gv = jnp.min(dist, axis=1, keepdims=True)
        gidx = jnp.min(jnp.where(dist == gv, ii, sentinel), axis=1,
                       keepdims=True)
        return gv, gidx

    def lexmin(p, q):
        pv, pi = p
        qv, qi = q
        take_q = (qv < pv) | ((qv == pv) & (qi < pi))
        return jnp.where(take_q, qv, pv), jnp.where(take_q, qi, pi)

    b1a = _GROUP_B1 // 128 * 128
    b1b = b1a + 128
    b2a = _GROUP_B2 // 128 * 128
    b2b = b2a + 128
    v1, i1 = lexmin(champ_slab(0, b1a),
                    champ_masked(b1a, b1b, b1a, _GROUP_B1))
    v2, i2 = lexmin(lexmin(champ_masked(b1a, b1b, _GROUP_B1, b1b),
                           champ_slab(b1b, b2a)),
                    champ_masked(b2a, b2b, b2a, _GROUP_B2))
    v3, i3 = lexmin(champ_masked(b2a, b2b, _GROUP_B2, b2b),
                    champ_slab(b2b, k_total))

    t1 = v1.astype(jnp.bfloat16).astype(jnp.float32)
    take2 = (v2 < t1) | ((v2 == t1) & (i2 < i1))
    acc_v = jnp.where(take2, v2, t1)
    acc_i = jnp.where(take2, i2, i1)
    t2 = acc_v.astype(jnp.bfloat16).astype(jnp.float32)
    take3 = (v3 < t2) | ((v3 == t2) & (i3 < acc_i))
    idx_ref[...] = jnp.where(take3, i3, acc_i)


def _out_body(flat_ref, q_ref, wfT_ref, bf_ref, o_ref):
    q16 = q_ref[...].astype(jnp.bfloat16)
    qo = jax.lax.dot_general(
        q16, wfT_ref[...], (((1,), (0,)), ((), ())),
        preferred_element_type=jnp.float32) + bf_ref[...]
    fl = flat_ref[...]
    o_ref[...] = fl + (qo - fl)


def _sc_gather(table, idx2d, n, d_code):
    mesh = plsc.VectorSubcoreMesh(core_axis_name="core", subcore_axis_name="subcore")

    @pl.kernel(out_type=jax.ShapeDtypeStruct((n, d_code), table.dtype), mesh=mesh)
    def gather_kernel(x_hbm, i_hbm, o_hbm):
        def body(i_vmem, o_vmem):
            pltpu.sync_copy(x_hbm.at[i_vmem.at[0]], o_vmem)

        pltpu.emit_pipeline(
            body,
            grid=(n // _GATHER_W,),
            in_specs=[pl.BlockSpec((1, _GATHER_W), index_map=lambda i: (0, i))],
            out_specs=[pl.BlockSpec((_GATHER_W, d_code), index_map=lambda i: (i, 0))],
            core_axis_name=("core", "subcore"),
            dimension_semantics=(pltpu.PARALLEL,),
        )(i_hbm, o_hbm)

    return gather_kernel(table, idx2d)


def kernel(features, codebook, Wt, bt, Wf, bf):
    orig_shape = features.shape
    d_in = orig_shape[-1]
    flat = features.reshape(-1, d_in)
    n = flat.shape[0]
    k, d_code = codebook.shape

    wtT = Wt.T
    cbT = codebook.T
    wfT = Wf.T
    bt2 = bt.reshape(1, d_code)
    bf2 = bf.reshape(1, d_in)

    nsteps = n // _M_BLK
    idx2d = pl.pallas_call(
        _argmin_body,
        grid=(nsteps,),
        in_specs=[
            pl.BlockSpec((_M_BLK, d_in), lambda i: (i, 0)),
            pl.BlockSpec((d_in, d_code), lambda i: (0, 0)),
            pl.BlockSpec((1, d_code), lambda i: (0, 0)),
            pl.BlockSpec((d_code, k), lambda i: (0, 0)),
        ],
        out_specs=pl.BlockSpec((_M_BLK, 1), lambda i: (i, 0)),
        out_shape=jax.ShapeDtypeStruct((n, 1), jnp.int32),
        scratch_shapes=[pltpu.VMEM((1, k), jnp.float32)],
        compiler_params=pltpu.CompilerParams(
            dimension_semantics=("arbitrary",)),
    )(flat, wtT, bt2, cbT)

    indices = idx2d[:, 0]
    quantized = _sc_gather(codebook, indices.reshape(1, n), n, d_code)

    out = pl.pallas_call(
        _out_body,
        grid=(n // 1024,),
        in_specs=[
            pl.BlockSpec((1024, d_in), lambda i: (i, 0)),
            pl.BlockSpec((1024, d_code), lambda i: (i, 0)),
            pl.BlockSpec((d_code, d_in), lambda i: (0, 0)),
            pl.BlockSpec((1, d_in), lambda i: (0, 0)),
        ],
        out_specs=pl.BlockSpec((1024, d_in), lambda i: (i, 0)),
        out_shape=jax.ShapeDtypeStruct((n, d_in), jnp.float32),
        compiler_params=pltpu.CompilerParams(
            dimension_semantics=("arbitrary",)),
    )(flat, quantized, wfT, bf2)

    return out.reshape(orig_shape), indices.reshape(orig_shape[:-1])
